# Initial kernel scaffold; baseline (speedup 1.0000x reference)
import jax
import jax.numpy as jnp
from jax.experimental import pallas as pl


def kernel(x, edge_index, W1, b1, W2, b2, Wl1, bl1, Wl2, bl2):
    raise NotImplementedError("write your pallas kernel here")


# speedup vs baseline: 8.8227x; 8.8227x over previous
import functools

import jax
import jax.numpy as jnp
from jax import lax
from jax.experimental import pallas as pl
from jax.experimental.pallas import tpu as pltpu
from jax.experimental.pallas import tpu_sc as plsc

N = 10000
E = 320000
D_IN = 128
D_HID = 64

NC = 2
NS = 16
NW = NC * NS
CHUNK = 128
NCHUNKS = E // CHUNK
ROWS_PER_SUB = 624
TAIL_ROWS = N - NS * ROWS_PER_SUB
DEG_W = 16

_BASE_CHUNKS = NCHUNKS // NW
_EXTRA_WORKERS = NCHUNKS % NW


def _n_chunks(wid):
    return _BASE_CHUNKS + (wid < _EXTRA_WORKERS).astype(jnp.int32)


def _worker_id():
    return lax.axis_index("s") * NC + lax.axis_index("c")


def _copy_share(src, dst, s):
    r0 = s * ROWS_PER_SUB
    pltpu.sync_copy(src.at[pl.ds(r0, ROWS_PER_SUB)],
                    dst.at[pl.ds(r0, ROWS_PER_SUB)])

    @pl.when(s == NS - 1)
    def _():
        t0 = NS * ROWS_PER_SUB
        pltpu.sync_copy(src.at[pl.ds(t0, TAIL_ROWS)],
                        dst.at[pl.ds(t0, TAIL_ROWS)])


def _writeback(acc, out0, out1, c, s):
    @pl.when(c == 0)
    def _():
        _copy_share(acc, out0, s)

    @pl.when(c == 1)
    def _():
        _copy_share(acc, out1, s)


@functools.lru_cache(maxsize=None)
def _sc_mesh():
    return plsc.VectorSubcoreMesh(
        core_axis_name="c", subcore_axis_name="s", num_cores=NC, num_subcores=NS
    )


@functools.lru_cache(maxsize=None)
def _deg_sc():
    @functools.partial(
        pl.kernel,
        out_type=(
            jax.ShapeDtypeStruct((N, DEG_W), jnp.float32),
            jax.ShapeDtypeStruct((N, DEG_W), jnp.float32),
        ),
        mesh=_sc_mesh(),
        scratch_types=[
            pltpu.VMEM((CHUNK,), jnp.int32),
            pltpu.VMEM((CHUNK, DEG_W), jnp.float32),
            pltpu.VMEM_SHARED((N, DEG_W), jnp.float32),
        ],
        compiler_params=pltpu.CompilerParams(use_tc_tiling_on_sc=False),
    )
    def deg_kernel(dst_hbm, ones_hbm, zeros_hbm, out0, out1, didx_v, ones_v, acc):
        c = lax.axis_index("c")
        s = lax.axis_index("s")
        wid = _worker_id()
        _copy_share(zeros_hbm, acc, s)
        pltpu.sync_copy(ones_hbm, ones_v)
        plsc.subcore_barrier()

        def body(i, carry):
            base = (wid + i * NW) * CHUNK
            pltpu.sync_copy(dst_hbm.at[pl.ds(base, CHUNK)], didx_v)
            pltpu.sync_copy(ones_v, acc.at[didx_v], add=True)
            return carry

        lax.fori_loop(0, _n_chunks(wid), body, 0)
        plsc.subcore_barrier()
        _writeback(acc, out0, out1, c, s)

    return deg_kernel


@functools.lru_cache(maxsize=None)
def _msg_sc():
    @functools.partial(
        pl.kernel,
        out_type=(
            jax.ShapeDtypeStruct((N, D_HID), jnp.float32),
            jax.ShapeDtypeStruct((N, D_HID), jnp.float32),
        ),
        mesh=_sc_mesh(),
        scratch_types=[
            pltpu.VMEM((CHUNK,), jnp.int32),
            pltpu.VMEM((CHUNK,), jnp.int32),
            pltpu.VMEM((CHUNK, D_HID), jnp.float32),
            pltpu.VMEM_SHARED((N, D_HID), jnp.float32),
            pltpu.SemaphoreType.DMA,
        ],
        compiler_params=pltpu.CompilerParams(use_tc_tiling_on_sc=False),
    )
    def msg_kernel(y_hbm, src_hbm, dst_hbm, zeros_hbm, out0, out1,
                   sidx_v, didx_v, rows_v, acc, sem):
        c = lax.axis_index("c")
        s = lax.axis_index("s")
        wid = _worker_id()
        _copy_share(zeros_hbm, acc, s)
        plsc.subcore_barrier()

        def body(i, carry):
            base = (wid + i * NW) * CHUNK
            pltpu.sync_copy(src_hbm.at[pl.ds(base, CHUNK)], sidx_v)
            pltpu.sync_copy(dst_hbm.at[pl.ds(base, CHUNK)], didx_v)
            pltpu.async_copy(y_hbm.at[sidx_v], rows_v, sem).wait()
            pltpu.sync_copy(rows_v, acc.at[didx_v], add=True)
            return carry

        lax.fori_loop(0, _n_chunks(wid), body, 0)
        plsc.subcore_barrier()
        _writeback(acc, out0, out1, c, s)

    return msg_kernel


@functools.lru_cache(maxsize=None)
def _pairs_sc():
    @functools.partial(
        pl.kernel,
        out_type=(
            jax.ShapeDtypeStruct((E, D_HID), jnp.float32),
            jax.ShapeDtypeStruct((E, D_HID), jnp.float32),
        ),
        mesh=_sc_mesh(),
        scratch_types=[
            pltpu.VMEM((CHUNK,), jnp.int32),
            pltpu.VMEM((CHUNK,), jnp.int32),
            pltpu.VMEM((CHUNK, D_HID), jnp.float32),
            pltpu.VMEM((CHUNK, D_HID), jnp.float32),
            pltpu.SemaphoreType.DMA,
            pltpu.SemaphoreType.DMA,
        ],
        compiler_params=pltpu.CompilerParams(use_tc_tiling_on_sc=False),
    )
    def pairs_kernel(ha_hbm, hb_hbm, src_hbm, dst_hbm, pa_out, pb_out,
                     sidx_v, didx_v, rowsa_v, rowsb_v, sema, semb):
        wid = _worker_id()

        def body(i, carry):
            base = (wid + i * NW) * CHUNK
            pltpu.sync_copy(src_hbm.at[pl.ds(base, CHUNK)], sidx_v)
            pltpu.sync_copy(dst_hbm.at[pl.ds(base, CHUNK)], didx_v)
            cpa = pltpu.async_copy(ha_hbm.at[sidx_v], rowsa_v, sema)
            cpb = pltpu.async_copy(hb_hbm.at[didx_v], rowsb_v, semb)
            cpa.wait()
            cpb.wait()
            pltpu.sync_copy(rowsa_v, pa_out.at[pl.ds(base, CHUNK)])
            pltpu.sync_copy(rowsb_v, pb_out.at[pl.ds(base, CHUNK)])
            return carry

        lax.fori_loop(0, _n_chunks(wid), body, 0)

    return pairs_kernel


_MBLK = 2000
_EBLK = 4000


def _dis_block(d0_ref, d1_ref):
    deg = d0_ref[:, 0:1] + d1_ref[:, 0:1] + 1.0
    return lax.rsqrt(deg)


def _mm1_body(x_ref, w_ref, d0_ref, d1_ref, o_ref):
    dis = _dis_block(d0_ref, d1_ref)
    xw = jnp.dot(x_ref[...], w_ref[...], preferred_element_type=jnp.float32)
    o_ref[...] = xw * dis


def _combine1_body(s0_ref, s1_ref, y_ref, d0_ref, d1_ref, b_ref, w_ref, o_ref):
    dis = _dis_block(d0_ref, d1_ref)
    h = jnp.maximum((s0_ref[...] + s1_ref[...] + y_ref[...]) * dis + b_ref[...], 0.0)
    o_ref[...] = jnp.dot(h, w_ref[...], preferred_element_type=jnp.float32) * dis


def _combine2_body(s0_ref, s1_ref, y_ref, d0_ref, d1_ref, b_ref, wl1_ref,
                   bl1_ref, oa_ref, ob_ref):
    dis = _dis_block(d0_ref, d1_ref)
    h = jnp.maximum((s0_ref[...] + s1_ref[...] + y_ref[...]) * dis + b_ref[...], 0.0)
    oa_ref[...] = (jnp.dot(h, wl1_ref[0:D_HID, :], preferred_element_type=jnp.float32)
                   + bl1_ref[...])
    ob_ref[...] = jnp.dot(h, wl1_ref[D_HID:2 * D_HID, :],
                          preferred_element_type=jnp.float32)


def _head_body(a_ref, b_ref, w_ref, bl_ref, o_ref):
    r = jnp.maximum(a_ref[...] + b_ref[...], 0.0)
    z = jnp.dot(r, w_ref[...], preferred_element_type=jnp.float32) + bl_ref[...]
    m = jnp.max(z, axis=1, keepdims=True)
    lse = m + jnp.log(jnp.sum(jnp.exp(z - m), axis=1, keepdims=True))
    o_ref[...] = z - lse


def _node_spec(width):
    return pl.BlockSpec((_MBLK, width), lambda i: (i, 0))


def _full_spec(shape):
    return pl.BlockSpec(shape, lambda i: tuple(0 for _ in shape))


def kernel(x, edge_index, W1, b1, W2, b2, Wl1, bl1, Wl2, bl2):
    src = edge_index[0].astype(jnp.int32)
    dst = edge_index[1].astype(jnp.int32)

    zeros_deg = jnp.zeros((N, DEG_W), jnp.float32)
    ones_deg = jnp.ones((CHUNK, DEG_W), jnp.float32)
    zeros_hid = jnp.zeros((N, D_HID), jnp.float32)
    b1r = b1.reshape(1, D_HID)
    b2r = b2.reshape(1, D_HID)
    bl1r = bl1.reshape(1, D_HID)
    bl2r = bl2.reshape(1, 2)

    deg0, deg1 = _deg_sc()(dst, ones_deg, zeros_deg)

    y1 = pl.pallas_call(
        _mm1_body,
        grid=(N // _MBLK,),
        in_specs=[
            _node_spec(D_IN),
            _full_spec((D_IN, D_HID)),
            _node_spec(DEG_W),
            _node_spec(DEG_W),
        ],
        out_specs=_node_spec(D_HID),
        out_shape=jax.ShapeDtypeStruct((N, D_HID), jnp.float32),
    )(x, W1, deg0, deg1)

    s1a, s1b = _msg_sc()(y1, src, dst, zeros_hid)

    y2 = pl.pallas_call(
        _combine1_body,
        grid=(N // _MBLK,),
        in_specs=[
            _node_spec(D_HID),
            _node_spec(D_HID),
            _node_spec(D_HID),
            _node_spec(DEG_W),
            _node_spec(DEG_W),
            _full_spec((1, D_HID)),
            _full_spec((D_HID, D_HID)),
        ],
        out_specs=_node_spec(D_HID),
        out_shape=jax.ShapeDtypeStruct((N, D_HID), jnp.float32),
    )(s1a, s1b, y1, deg0, deg1, b1r, W2)

    s2a, s2b = _msg_sc()(y2, src, dst, zeros_hid)

    ha, hb = pl.pallas_call(
        _combine2_body,
        grid=(N // _MBLK,),
        in_specs=[
            _node_spec(D_HID),
            _node_spec(D_HID),
            _node_spec(D_HID),
            _node_spec(DEG_W),
            _node_spec(DEG_W),
            _full_spec((1, D_HID)),
            _full_spec((D_IN, D_HID)),
            _full_spec((1, D_HID)),
        ],
        out_specs=(_node_spec(D_HID), _node_spec(D_HID)),
        out_shape=(
            jax.ShapeDtypeStruct((N, D_HID), jnp.float32),
            jax.ShapeDtypeStruct((N, D_HID), jnp.float32),
        ),
    )(s2a, s2b, y2, deg0, deg1, b2r, Wl1, bl1r)

    pa, pb = _pairs_sc()(ha, hb, src, dst)

    out = pl.pallas_call(
        _head_body,
        grid=(E // _EBLK,),
        in_specs=[
            pl.BlockSpec((_EBLK, D_HID), lambda i: (i, 0)),
            pl.BlockSpec((_EBLK, D_HID), lambda i: (i, 0)),
            _full_spec((D_HID, 2)),
            _full_spec((1, 2)),
        ],
        out_specs=pl.BlockSpec((_EBLK, 2), lambda i: (i, 0)),
        out_shape=jax.ShapeDtypeStruct((E, 2), jnp.float32),
    )(pa, pb, Wl2, bl2r)

    return out


# speedup vs baseline: 12.1261x; 1.3744x over previous
import functools

import jax
import jax.numpy as jnp
from jax import lax
from jax.experimental import pallas as pl
from jax.experimental.pallas import tpu as pltpu
from jax.experimental.pallas import tpu_sc as plsc

N = 10000
E = 320000
D_IN = 128
D_HID = 64

NC = 2
NS = 16
NW = NC * NS
CHUNK = 128
NLOC = 80
NCHUNKS_PAD = NW * NLOC
E_PAD = NCHUNKS_PAD * CHUNK
PAD_BIN = 16
N_ACC = N + PAD_BIN
ROWS_PER_SUB = 624
TAIL_ROWS = N - NS * ROWS_PER_SUB
DEG_W = 16

G_MSG = 4
NG_MSG = NLOC // G_MSG
G_PAIR = 2
NG_PAIR = NLOC // G_PAIR


def _worker_id():
    return lax.axis_index("s") * NC + lax.axis_index("c")


def _copy_share(src, dst, s):
    r0 = s * ROWS_PER_SUB
    pltpu.sync_copy(src.at[pl.ds(r0, ROWS_PER_SUB)],
                    dst.at[pl.ds(r0, ROWS_PER_SUB)])

    @pl.when(s == NS - 1)
    def _():
        t0 = NS * ROWS_PER_SUB
        pltpu.sync_copy(src.at[pl.ds(t0, TAIL_ROWS)],
                        dst.at[pl.ds(t0, TAIL_ROWS)])


def _writeback(acc, out0, out1, c, s):
    @pl.when(c == 0)
    def _():
        _copy_share(acc, out0, s)

    @pl.when(c == 1)
    def _():
        _copy_share(acc, out1, s)


@functools.lru_cache(maxsize=None)
def _sc_mesh():
    return plsc.VectorSubcoreMesh(
        core_axis_name="c", subcore_axis_name="s", num_cores=NC, num_subcores=NS
    )


@functools.lru_cache(maxsize=None)
def _deg_sc():
    @functools.partial(
        pl.kernel,
        out_type=(
            jax.ShapeDtypeStruct((N, DEG_W), jnp.float32),
            jax.ShapeDtypeStruct((N, DEG_W), jnp.float32),
        ),
        mesh=_sc_mesh(),
        scratch_types=[
            pltpu.VMEM((NLOC, CHUNK), jnp.int32),
            pltpu.VMEM((CHUNK, DEG_W), jnp.float32),
            pltpu.VMEM_SHARED((N_ACC, DEG_W), jnp.float32),
            pltpu.SemaphoreType.DMA,
        ],
        compiler_params=pltpu.CompilerParams(use_tc_tiling_on_sc=False),
    )
    def deg_kernel(dst2d_hbm, ones_hbm, zeros_hbm, out0, out1,
                   didx, ones_v, acc, sem):
        c = lax.axis_index("c")
        s = lax.axis_index("s")
        w = _worker_id()
        c0 = pl.multiple_of(w * NLOC, 8)
        pltpu.sync_copy(dst2d_hbm.at[pl.ds(c0, NLOC)], didx)
        pltpu.sync_copy(ones_hbm, ones_v)
        _copy_share(zeros_hbm, acc, s)
        plsc.subcore_barrier()

        def fire16(t, carry):
            for j in range(16):
                li = t * 16 + j
                pltpu.make_async_copy(ones_v, acc.at[didx.at[li]], sem).start(add=True)
            return carry

        def drain16(t, carry):
            for j in range(16):
                li = t * 16 + j
                pltpu.make_async_copy(ones_v, acc.at[didx.at[li]], sem).wait()
            return carry

        lax.fori_loop(0, NLOC // 16, fire16, 0)
        lax.fori_loop(0, NLOC // 16, drain16, 0)
        plsc.subcore_barrier()
        _writeback(acc, out0, out1, c, s)

    return deg_kernel


@functools.lru_cache(maxsize=None)
def _msg_sc():
    @functools.partial(
        pl.kernel,
        out_type=(
            jax.ShapeDtypeStruct((N, D_HID), jnp.float32),
            jax.ShapeDtypeStruct((N, D_HID), jnp.float32),
        ),
        mesh=_sc_mesh(),
        scratch_types=[
            pltpu.VMEM((NLOC * CHUNK,), jnp.int32),
            pltpu.VMEM((NLOC, CHUNK), jnp.int32),
            pltpu.VMEM((2 * G_MSG, CHUNK, D_HID), jnp.float32),
            pltpu.VMEM_SHARED((N_ACC, D_HID), jnp.float32),
            pltpu.SemaphoreType.DMA,
            pltpu.SemaphoreType.DMA,
            pltpu.SemaphoreType.DMA,
            pltpu.SemaphoreType.DMA,
        ],
        compiler_params=pltpu.CompilerParams(use_tc_tiling_on_sc=False),
    )
    def msg_kernel(y_hbm, src1d_hbm, dst2d_hbm, zeros_hbm, out0, out1,
                   sidx, didx, rows, acc, sga, sgb, ssa, ssb):
        c = lax.axis_index("c")
        s = lax.axis_index("s")
        w = _worker_id()
        c0 = pl.multiple_of(w * NLOC, 8)
        pltpu.sync_copy(src1d_hbm.at[pl.ds(c0 * CHUNK, NLOC * CHUNK)], sidx)
        pltpu.sync_copy(dst2d_hbm.at[pl.ds(c0, NLOC)], didx)
        _copy_share(zeros_hbm, acc, s)

        def gather_desc(g, bank, j, sem):
            li = g * G_MSG + j
            return pltpu.make_async_copy(
                y_hbm.at[sidx.at[pl.ds(li * CHUNK, CHUNK)]],
                rows.at[bank * G_MSG + j], sem)

        def scatter_desc(g, bank, j, sem):
            li = g * G_MSG + j
            return pltpu.make_async_copy(
                rows.at[bank * G_MSG + j], acc.at[didx.at[li]], sem)

        def fire_gathers(g, bank, sem):
            for j in range(G_MSG):
                gather_desc(g, bank, j, sem).start()

        def drain_gathers(g, bank, sem):
            for j in range(G_MSG):
                gather_desc(g, bank, j, sem).wait()

        def fire_scatters(g, bank, sem):
            for j in range(G_MSG):
                scatter_desc(g, bank, j, sem).start(add=True)

        def drain_scatters(g, bank, sem):
            for j in range(G_MSG):
                scatter_desc(g, bank, j, sem).wait()

        fire_gathers(0, 0, sga)
        fire_gathers(1, 1, sgb)
        plsc.subcore_barrier()

        def body(t, carry):
            g0 = 2 * t
            g1 = g0 + 1
            drain_gathers(g0, 0, sga)
            fire_scatters(g0, 0, ssa)
            drain_gathers(g1, 1, sgb)
            fire_scatters(g1, 1, ssb)
            drain_scatters(g0, 0, ssa)
            fire_gathers(g0 + 2, 0, sga)
            drain_scatters(g1, 1, ssb)
            fire_gathers(g1 + 2, 1, sgb)
            return carry

        lax.fori_loop(0, NG_MSG // 2 - 1, body, 0)
        g0 = NG_MSG - 2
        g1 = NG_MSG - 1
        drain_gathers(g0, 0, sga)
        fire_scatters(g0, 0, ssa)
        drain_gathers(g1, 1, sgb)
        fire_scatters(g1, 1, ssb)
        drain_scatters(g0, 0, ssa)
        drain_scatters(g1, 1, ssb)
        plsc.subcore_barrier()
        _writeback(acc, out0, out1, c, s)

    return msg_kernel


@functools.lru_cache(maxsize=None)
def _pairs_sc():
    @functools.partial(
        pl.kernel,
        out_type=(
            jax.ShapeDtypeStruct((E_PAD, D_HID), jnp.float32),
            jax.ShapeDtypeStruct((E_PAD, D_HID), jnp.float32),
        ),
        mesh=_sc_mesh(),
        scratch_types=[
            pltpu.VMEM((NLOC * CHUNK,), jnp.int32),
            pltpu.VMEM((NLOC * CHUNK,), jnp.int32),
            pltpu.VMEM((2 * G_PAIR, CHUNK, D_HID), jnp.float32),
            pltpu.VMEM((2 * G_PAIR, CHUNK, D_HID), jnp.float32),
            pltpu.SemaphoreType.DMA,
            pltpu.SemaphoreType.DMA,
            pltpu.SemaphoreType.DMA,
            pltpu.SemaphoreType.DMA,
        ],
        compiler_params=pltpu.CompilerParams(use_tc_tiling_on_sc=False),
    )
    def pairs_kernel(ha_hbm, hb_hbm, src1d_hbm, dst1d_hbm, pa_out, pb_out,
                     sidx, didx, rowsa, rowsb, sga, sgb, swa, swb):
        w = _worker_id()
        c0 = pl.multiple_of(w * NLOC, 8)
        pltpu.sync_copy(src1d_hbm.at[pl.ds(c0 * CHUNK, NLOC * CHUNK)], sidx)
        pltpu.sync_copy(dst1d_hbm.at[pl.ds(c0 * CHUNK, NLOC * CHUNK)], didx)

        def gdescs(g, bank, j, sem):
            li = g * G_PAIR + j
            slot = bank * G_PAIR + j
            return (
                pltpu.make_async_copy(
                    ha_hbm.at[sidx.at[pl.ds(li * CHUNK, CHUNK)]],
                    rowsa.at[slot], sem),
                pltpu.make_async_copy(
                    hb_hbm.at[didx.at[pl.ds(li * CHUNK, CHUNK)]],
                    rowsb.at[slot], sem),
            )

        def wdescs(g, bank, j, sem):
            li = g * G_PAIR + j
            slot = bank * G_PAIR + j
            base = (c0 + li) * CHUNK
            return (
                pltpu.make_async_copy(rowsa.at[slot],
                                      pa_out.at[pl.ds(base, CHUNK)], sem),
                pltpu.make_async_copy(rowsb.at[slot],
                                      pb_out.at[pl.ds(base, CHUNK)], sem),
            )

        def fire(descfn, g, bank, sem):
            for j in range(G_PAIR):
                for d in descfn(g, bank, j, sem):
                    d.start()

        def drain(descfn, g, bank, sem):
            for j in range(G_PAIR):
                for d in descfn(g, bank, j, sem):
                    d.wait()

        fire(gdescs, 0, 0, sga)
        fire(gdescs, 1, 1, sgb)

        def body(t, carry):
            g0 = 2 * t
            g1 = g0 + 1
            drain(gdescs, g0, 0, sga)
            fire(wdescs, g0, 0, swa)
            drain(gdescs, g1, 1, sgb)
            fire(wdescs, g1, 1, swb)
            drain(wdescs, g0, 0, swa)
            fire(gdescs, g0 + 2, 0, sga)
            drain(wdescs, g1, 1, swb)
            fire(gdescs, g1 + 2, 1, sgb)
            return carry

        lax.fori_loop(0, NG_PAIR // 2 - 1, body, 0)
        g0 = NG_PAIR - 2
        g1 = NG_PAIR - 1
        drain(gdescs, g0, 0, sga)
        fire(wdescs, g0, 0, swa)
        drain(gdescs, g1, 1, sgb)
        fire(wdescs, g1, 1, swb)
        drain(wdescs, g0, 0, swa)
        drain(wdescs, g1, 1, swb)

    return pairs_kernel


_MBLK = 2000
_EBLK = 4000


def _dis_block(d0_ref, d1_ref):
    deg = d0_ref[:, 0:1] + d1_ref[:, 0:1] + 1.0
    return lax.rsqrt(deg)


def _mm1_body(x_ref, w_ref, d0_ref, d1_ref, o_ref):
    dis = _dis_block(d0_ref, d1_ref)
    xw = jnp.dot(x_ref[...], w_ref[...], preferred_element_type=jnp.float32)
    o_ref[...] = xw * dis


def _combine1_body(s0_ref, s1_ref, y_ref, d0_ref, d1_ref, b_ref, w_ref, o_ref):
    dis = _dis_block(d0_ref, d1_ref)
    h = jnp.maximum((s0_ref[...] + s1_ref[...] + y_ref[...]) * dis + b_ref[...], 0.0)
    o_ref[...] = jnp.dot(h, w_ref[...], preferred_element_type=jnp.float32) * dis


def _combine2_body(s0_ref, s1_ref, y_ref, d0_ref, d1_ref, b_ref, wl1_ref,
                   bl1_ref, oa_ref, ob_ref):
    dis = _dis_block(d0_ref, d1_ref)
    h = jnp.maximum((s0_ref[...] + s1_ref[...] + y_ref[...]) * dis + b_ref[...], 0.0)
    oa_ref[...] = (jnp.dot(h, wl1_ref[0:D_HID, :], preferred_element_type=jnp.float32)
                   + bl1_ref[...])
    ob_ref[...] = jnp.dot(h, wl1_ref[D_HID:2 * D_HID, :],
                          preferred_element_type=jnp.float32)


def _head_body(a_ref, b_ref, w_ref, bl_ref, o_ref):
    r = jnp.maximum(a_ref[...] + b_ref[...], 0.0)
    z = jnp.dot(r, w_ref[...], preferred_element_type=jnp.float32) + bl_ref[...]
    m = jnp.max(z, axis=1, keepdims=True)
    lse = m + jnp.log(jnp.sum(jnp.exp(z - m), axis=1, keepdims=True))
    o_ref[...] = z - lse


def _node_spec(width):
    return pl.BlockSpec((_MBLK, width), lambda i: (i, 0))


def _full_spec(shape):
    return pl.BlockSpec(shape, lambda i: tuple(0 for _ in shape))


def kernel(x, edge_index, W1, b1, W2, b2, Wl1, bl1, Wl2, bl2):
    src = edge_index[0].astype(jnp.int32)
    dst = edge_index[1].astype(jnp.int32)

    npad = E_PAD - E
    pad_src = (jnp.arange(npad, dtype=jnp.int32) * 37) % N
    pad_dst = N + (jnp.arange(npad, dtype=jnp.int32) % PAD_BIN)
    srcp = jnp.concatenate([src, pad_src])
    dstp = jnp.concatenate([dst, pad_dst])
    dstp2d = dstp.reshape(NCHUNKS_PAD, CHUNK)

    zeros_deg = jnp.zeros((N, DEG_W), jnp.float32)
    ones_deg = jnp.ones((CHUNK, DEG_W), jnp.float32)
    zeros_hid = jnp.zeros((N, D_HID), jnp.float32)
    b1r = b1.reshape(1, D_HID)
    b2r = b2.reshape(1, D_HID)
    bl1r = bl1.reshape(1, D_HID)
    bl2r = bl2.reshape(1, 2)

    deg0, deg1 = _deg_sc()(dstp2d, ones_deg, zeros_deg)

    y1 = pl.pallas_call(
        _mm1_body,
        grid=(N // _MBLK,),
        in_specs=[
            _node_spec(D_IN),
            _full_spec((D_IN, D_HID)),
            _node_spec(DEG_W),
            _node_spec(DEG_W),
        ],
        out_specs=_node_spec(D_HID),
        out_shape=jax.ShapeDtypeStruct((N, D_HID), jnp.float32),
    )(x, W1, deg0, deg1)

    s1a, s1b = _msg_sc()(y1, srcp, dstp2d, zeros_hid)

    y2 = pl.pallas_call(
        _combine1_body,
        grid=(N // _MBLK,),
        in_specs=[
            _node_spec(D_HID),
            _node_spec(D_HID),
            _node_spec(D_HID),
            _node_spec(DEG_W),
            _node_spec(DEG_W),
            _full_spec((1, D_HID)),
            _full_spec((D_HID, D_HID)),
        ],
        out_specs=_node_spec(D_HID),
        out_shape=jax.ShapeDtypeStruct((N, D_HID), jnp.float32),
    )(s1a, s1b, y1, deg0, deg1, b1r, W2)

    s2a, s2b = _msg_sc()(y2, srcp, dstp2d, zeros_hid)

    ha, hb = pl.pallas_call(
        _combine2_body,
        grid=(N // _MBLK,),
        in_specs=[
            _node_spec(D_HID),
            _node_spec(D_HID),
            _node_spec(D_HID),
            _node_spec(DEG_W),
            _node_spec(DEG_W),
            _full_spec((1, D_HID)),
            _full_spec((D_IN, D_HID)),
            _full_spec((1, D_HID)),
        ],
        out_specs=(_node_spec(D_HID), _node_spec(D_HID)),
        out_shape=(
            jax.ShapeDtypeStruct((N, D_HID), jnp.float32),
            jax.ShapeDtypeStruct((N, D_HID), jnp.float32),
        ),
    )(s2a, s2b, y2, deg0, deg1, b2r, Wl1, bl1r)

    pa, pb = _pairs_sc()(ha, hb, srcp, dstp)

    out = pl.pallas_call(
        _head_body,
        grid=(E // _EBLK,),
        in_specs=[
            pl.BlockSpec((_EBLK, D_HID), lambda i: (i, 0)),
            pl.BlockSpec((_EBLK, D_HID), lambda i: (i, 0)),
            _full_spec((D_HID, 2)),
            _full_spec((1, 2)),
        ],
        out_specs=pl.BlockSpec((_EBLK, 2), lambda i: (i, 0)),
        out_shape=jax.ShapeDtypeStruct((E, 2), jnp.float32),
    )(pa, pb, Wl2, bl2r)

    return out
